# baseline (speedup 1.0000x reference)
import jax
import jax.numpy as jnp
from jax import lax
from jax.experimental import pallas as pl
from jax.experimental.pallas import tpu as pltpu

_N = 16384
_C = 1000
_CW = 1024
_NBUF = 4
_NCHUNK = _N // _CW


def _chunk_sum(yt, yp):
    is_pos = yt == 1
    t = jnp.exp(yp)
    r = 1.0 / t
    s_neg = jnp.sum(jnp.where(is_pos, 0.0, t), axis=0)
    s_pos = jnp.sum(jnp.where(is_pos, r, 0.0), axis=0)
    return jnp.sum(jnp.log1p(s_neg * s_pos))


def _body(yt_hbm, yp_hbm, out_ref, yt_buf, yp_buf, yt_sem, yp_sem):
    def start(chunk, slot):
        pltpu.make_async_copy(
            yt_hbm.at[:, pl.ds(chunk * _CW, _CW)], yt_buf.at[slot], yt_sem.at[slot]
        ).start()
        pltpu.make_async_copy(
            yp_hbm.at[:, pl.ds(chunk * _CW, _CW)], yp_buf.at[slot], yp_sem.at[slot]
        ).start()

    for i in range(_NBUF):
        start(i, i)

    def step(i, acc):
        slot = lax.rem(i, _NBUF)
        pltpu.make_async_copy(
            yt_hbm.at[:, pl.ds(0, _CW)], yt_buf.at[slot], yt_sem.at[slot]
        ).wait()
        pltpu.make_async_copy(
            yp_hbm.at[:, pl.ds(0, _CW)], yp_buf.at[slot], yp_sem.at[slot]
        ).wait()
        cs = _chunk_sum(yt_buf[slot], yp_buf[slot])

        @pl.when(i + _NBUF < _NCHUNK)
        def _():
            start(i + _NBUF, slot)

        return acc + cs

    acc = lax.fori_loop(0, _NCHUNK, step, jnp.float32(0.0))
    out_ref[0, 0] = acc / _N


def kernel(y_true, y_pred):
    out = pl.pallas_call(
        _body,
        in_specs=[
            pl.BlockSpec(memory_space=pl.ANY),
            pl.BlockSpec(memory_space=pl.ANY),
        ],
        out_specs=pl.BlockSpec(memory_space=pltpu.SMEM),
        out_shape=jax.ShapeDtypeStruct((1, 1), jnp.float32),
        scratch_shapes=[
            pltpu.VMEM((_NBUF, _C, _CW), jnp.int32),
            pltpu.VMEM((_NBUF, _C, _CW), jnp.float32),
            pltpu.SemaphoreType.DMA((_NBUF,)),
            pltpu.SemaphoreType.DMA((_NBUF,)),
        ],
    )(y_true.T, y_pred.T)
    return out[0, 0].reshape(1)


# speedup vs baseline: 1.0339x; 1.0339x over previous
import jax
import jax.numpy as jnp
from jax import lax
from jax.experimental import pallas as pl
from jax.experimental.pallas import tpu as pltpu

_N = 16384
_C = 1000
_CW = 256
_NBUF = 16
_NCHUNK = _N // _CW


def _chunk_sum(yt, yp):
    is_pos = yt == 1
    t = jnp.exp(yp)
    r = 1.0 / t
    s_neg = jnp.sum(jnp.where(is_pos, 0.0, t), axis=0)
    s_pos = jnp.sum(jnp.where(is_pos, r, 0.0), axis=0)
    return jnp.sum(jnp.log1p(s_neg * s_pos))


def _body(yt_hbm, yp_hbm, out_ref, yt_buf, yp_buf, yt_sem, yp_sem):
    def start(chunk, slot):
        pltpu.make_async_copy(
            yt_hbm.at[:, pl.ds(chunk * _CW, _CW)], yt_buf.at[slot], yt_sem.at[slot]
        ).start()
        pltpu.make_async_copy(
            yp_hbm.at[:, pl.ds(chunk * _CW, _CW)], yp_buf.at[slot], yp_sem.at[slot]
        ).start()

    for i in range(_NBUF):
        start(i, i)

    def step(i, acc):
        slot = lax.rem(i, _NBUF)
        pltpu.make_async_copy(
            yt_hbm.at[:, pl.ds(0, _CW)], yt_buf.at[slot], yt_sem.at[slot]
        ).wait()
        pltpu.make_async_copy(
            yp_hbm.at[:, pl.ds(0, _CW)], yp_buf.at[slot], yp_sem.at[slot]
        ).wait()
        cs = _chunk_sum(yt_buf[slot], yp_buf[slot])

        @pl.when(i + _NBUF < _NCHUNK)
        def _():
            start(i + _NBUF, slot)

        return acc + cs

    acc = lax.fori_loop(0, _NCHUNK, step, jnp.float32(0.0))
    out_ref[0, 0] = acc / _N


def kernel(y_true, y_pred):
    out = pl.pallas_call(
        _body,
        in_specs=[
            pl.BlockSpec(memory_space=pl.ANY),
            pl.BlockSpec(memory_space=pl.ANY),
        ],
        out_specs=pl.BlockSpec(memory_space=pltpu.SMEM),
        out_shape=jax.ShapeDtypeStruct((1, 1), jnp.float32),
        scratch_shapes=[
            pltpu.VMEM((_NBUF, _C, _CW), jnp.int32),
            pltpu.VMEM((_NBUF, _C, _CW), jnp.float32),
            pltpu.SemaphoreType.DMA((_NBUF,)),
            pltpu.SemaphoreType.DMA((_NBUF,)),
        ],
    )(y_true.T, y_pred.T)
    return out[0, 0].reshape(1)
